# baseline (speedup 1.0000x reference)
import functools

import jax
import jax.numpy as jnp
from jax import lax
from jax.experimental import pallas as pl
from jax.experimental.pallas import tpu as pltpu
from jax.experimental.pallas import tpu_sc as plsc

N_ATOMS = 100000
N_FRAG = 200000
D_FEAT = 128
MID = 32
OUT_DIM = 3

NW = 32
CHUNK = 128
NPAD = 200704
RPW = NPAD // NW
NCH = RPW // CHUNK


def _tables_tc(x, G, bias96):
    BM = 2000

    def body(x_ref, g_ref, b_ref, p_ref, bb_ref):
        acc = jnp.dot(x_ref[...], g_ref[...], preferred_element_type=jnp.float32)
        acc = acc + b_ref[...]
        p_ref[...] = acc[:, :64].astype(jnp.bfloat16)
        bb_ref[...] = acc[:, 64:].astype(jnp.bfloat16)

    return pl.pallas_call(
        body,
        grid=(N_ATOMS // BM,),
        in_specs=[
            pl.BlockSpec((BM, D_FEAT), lambda i: (i, 0)),
            pl.BlockSpec((D_FEAT, 96), lambda i: (0, 0)),
            pl.BlockSpec((1, 96), lambda i: (0, 0)),
        ],
        out_specs=[
            pl.BlockSpec((BM, 64), lambda i: (i, 0)),
            pl.BlockSpec((BM, 32), lambda i: (i, 0)),
        ],
        out_shape=[
            jax.ShapeDtypeStruct((N_ATOMS, 64), jnp.bfloat16),
            jax.ShapeDtypeStruct((N_ATOMS, 32), jnp.bfloat16),
        ],
    )(x, G, bias96)


def _gather_sc(idx0, idx1, idx2, p64, b1t):
    HC = CHUNK // 2
    HPW = RPW // 2
    HOFF = NPAD // 2
    mesh = plsc.VectorSubcoreMesh(
        core_axis_name="c", subcore_axis_name="s", num_cores=2, num_subcores=16
    )

    @functools.partial(
        pl.kernel,
        out_type=jax.ShapeDtypeStruct((NPAD, 64), jnp.bfloat16),
        mesh=mesh,
        compiler_params=pltpu.CompilerParams(use_tc_tiling_on_sc=False),
        scratch_types=[
            pltpu.VMEM((HPW,), jnp.int32),
            pltpu.VMEM((HPW,), jnp.int32),
            pltpu.VMEM((HPW,), jnp.int32),
            pltpu.VMEM((HPW,), jnp.int32),
            pltpu.VMEM((HPW,), jnp.int32),
            pltpu.VMEM((HPW,), jnp.int32),
            pltpu.VMEM((2, HC, 64), jnp.bfloat16),
            pltpu.VMEM((2, HC, 64), jnp.bfloat16),
            pltpu.VMEM((2, HC, 64), jnp.bfloat16),
            pltpu.VMEM((2, HC, 64), jnp.bfloat16),
            pltpu.VMEM((2, HC, 32), jnp.bfloat16),
            pltpu.VMEM((2, HC, 32), jnp.bfloat16),
            pltpu.VMEM((2, CHUNK, 64), jnp.bfloat16),
            pltpu.SemaphoreType.DMA,
            pltpu.SemaphoreType.DMA,
            pltpu.SemaphoreType.DMA,
            pltpu.SemaphoreType.DMA,
        ],
    )
    def k(idx0_h, idx1_h, idx2_h, p_h, b_h, z_h,
          ib0l, ib0h, ib1l, ib1h, ib2l, ib2h,
          bp0l, bp0h, bp2l, bp2h, bbl, bbh, zb,
          ga, gb, wa, wb):
        wid = lax.axis_index("s") * 2 + lax.axis_index("c")
        lo = wid * HPW
        hi = HOFF + wid * HPW
        pltpu.sync_copy(idx0_h.at[pl.ds(lo, HPW)], ib0l)
        pltpu.sync_copy(idx0_h.at[pl.ds(hi, HPW)], ib0h)
        pltpu.sync_copy(idx1_h.at[pl.ds(lo, HPW)], ib1l)
        pltpu.sync_copy(idx1_h.at[pl.ds(hi, HPW)], ib1h)
        pltpu.sync_copy(idx2_h.at[pl.ds(lo, HPW)], ib2l)
        pltpu.sync_copy(idx2_h.at[pl.ds(hi, HPW)], ib2h)

        gsem = (ga, gb)
        wsem = (wa, wb)

        def issue(c, s):
            o = HC * c
            return (
                pltpu.async_copy(p_h.at[ib0l.at[pl.ds(o, HC)]], bp0l.at[s], gsem[s]),
                pltpu.async_copy(p_h.at[ib0h.at[pl.ds(o, HC)]], bp0h.at[s], gsem[s]),
                pltpu.async_copy(b_h.at[ib1l.at[pl.ds(o, HC)]], bbl.at[s], gsem[s]),
                pltpu.async_copy(b_h.at[ib1h.at[pl.ds(o, HC)]], bbh.at[s], gsem[s]),
                pltpu.async_copy(p_h.at[ib2l.at[pl.ds(o, HC)]], bp2l.at[s], gsem[s]),
                pltpu.async_copy(p_h.at[ib2h.at[pl.ds(o, HC)]], bp2h.at[s], gsem[s]),
            )

        def compute(s):
            halves = ((bp0l.at[s], bbl.at[s], bp2l.at[s], 0),
                      (bp0h.at[s], bbh.at[s], bp2h.at[s], 1))
            zr = zb.at[s]

            def row(r, _):
                for p0r, bbr, p2r, kk in halves:
                    b = bbr[r, pl.ds(0, 32)]
                    lo = p0r[r, pl.ds(0, 32)]
                    hi = p0r[r, pl.ds(32, 32)]
                    qlo = p2r[r, pl.ds(0, 32)]
                    qhi = p2r[r, pl.ds(32, 32)]
                    zr[2 * r + kk, pl.ds(0, 32)] = lo + b + qhi
                    zr[2 * r + kk, pl.ds(32, 32)] = hi + b + qlo
                return 0

            lax.fori_loop(0, HC, row, 0)

        gd = [None, None]
        wd = [None, None]
        zbase = wid * RPW
        gd[0] = issue(0, 0)
        for c in range(NCH):
            s = c & 1
            if c + 1 < NCH:
                gd[1 - s] = issue(c + 1, 1 - s)
            for d in gd[s]:
                d.wait()
            if wd[s] is not None:
                wd[s].wait()
            compute(s)
            wd[s] = pltpu.async_copy(
                zb.at[s], z_h.at[pl.ds(zbase + c * CHUNK, CHUNK)], wsem[s]
            )
        for s in (0, 1):
            if wd[s] is not None:
                wd[s].wait()

    return k(idx0, idx1, idx2, p64, b1t)


def _readout_tc(z128, w2t):
    BMZ = 2048
    nrows = z128.shape[0]

    def body(z_ref, w_ref, o_ref):
        z = jnp.maximum(z_ref[...], jnp.bfloat16(0.0))
        o_ref[...] = lax.dot_general(
            w_ref[...], z, (((1,), (1,)), ((), ())),
            preferred_element_type=jnp.float32,
        )

    return pl.pallas_call(
        body,
        grid=(nrows // BMZ,),
        in_specs=[
            pl.BlockSpec((BMZ, 128), lambda i: (i, 0)),
            pl.BlockSpec((6, 128), lambda i: (0, 0)),
        ],
        out_specs=pl.BlockSpec((6, BMZ), lambda i: (0, i)),
        out_shape=jax.ShapeDtypeStruct((6, nrows), jnp.float32),
    )(z128, w2t)


def kernel(x, frag_idx, W1, b1, Wout, bout):
    G = jnp.concatenate([W1[:D_FEAT], W1[2 * D_FEAT:], W1[D_FEAT:2 * D_FEAT]], axis=1)
    bias96 = jnp.concatenate([jnp.zeros((64,), jnp.float32), b1])[None, :]

    idx_pad = jnp.concatenate(
        [frag_idx, jnp.zeros((3, NPAD - N_FRAG), jnp.int32)], axis=1
    )

    p64, b1t = _tables_tc(x, G, bias96)
    z = _gather_sc(idx_pad[0], idx_pad[1], idx_pad[2], p64, b1t)

    m = jnp.concatenate([Wout, Wout], axis=0)
    w2t = jnp.kron(jnp.eye(2, dtype=jnp.float32), m).T
    w2t = w2t[jnp.array([0, 3, 1, 4, 2, 5])]
    w2t = w2t.astype(jnp.bfloat16)

    z128 = z.reshape(NPAD // 2, 128)
    out_t = _readout_tc(z128, w2t)
    eq_t = out_t.reshape(OUT_DIM, NPAD)
    return eq_t.T[:N_FRAG] + bout[None, :]


# speedup vs baseline: 1.0640x; 1.0640x over previous
import functools

import jax
import jax.numpy as jnp
from jax import lax
from jax.experimental import pallas as pl
from jax.experimental.pallas import tpu as pltpu
from jax.experimental.pallas import tpu_sc as plsc

N_ATOMS = 100000
N_FRAG = 200000
D_FEAT = 128
MID = 32
OUT_DIM = 3

NW = 32
CHUNK = 128
NPAD = 200704
RPW = NPAD // NW
NCH = RPW // CHUNK


def _tables_tc(x, G, bias96):
    BM = 2000

    def body(x_ref, g_ref, b_ref, p_ref, bb_ref):
        acc = jnp.dot(x_ref[...], g_ref[...], preferred_element_type=jnp.float32)
        acc = acc + b_ref[...]
        p_ref[...] = acc[:, :64]
        bb_ref[...] = acc[:, 64:]

    return pl.pallas_call(
        body,
        grid=(N_ATOMS // BM,),
        in_specs=[
            pl.BlockSpec((BM, D_FEAT), lambda i: (i, 0)),
            pl.BlockSpec((D_FEAT, 96), lambda i: (0, 0)),
            pl.BlockSpec((1, 96), lambda i: (0, 0)),
        ],
        out_specs=[
            pl.BlockSpec((BM, 64), lambda i: (i, 0)),
            pl.BlockSpec((BM, 32), lambda i: (i, 0)),
        ],
        out_shape=[
            jax.ShapeDtypeStruct((N_ATOMS, 64), jnp.float32),
            jax.ShapeDtypeStruct((N_ATOMS, 32), jnp.float32),
        ],
    )(x, G, bias96)


def _gather_sc(idx0, idx1, idx2, p64, b1t):
    HC = CHUNK // 2
    HPW = RPW // 2
    HOFF = NPAD // 2
    mesh = plsc.VectorSubcoreMesh(
        core_axis_name="c", subcore_axis_name="s", num_cores=2, num_subcores=16
    )

    @functools.partial(
        pl.kernel,
        out_type=jax.ShapeDtypeStruct((NPAD, 64), jnp.float32),
        mesh=mesh,
        compiler_params=pltpu.CompilerParams(use_tc_tiling_on_sc=False),
        scratch_types=[
            pltpu.VMEM((HPW,), jnp.int32),
            pltpu.VMEM((HPW,), jnp.int32),
            pltpu.VMEM((HPW,), jnp.int32),
            pltpu.VMEM((HPW,), jnp.int32),
            pltpu.VMEM((HPW,), jnp.int32),
            pltpu.VMEM((HPW,), jnp.int32),
            pltpu.VMEM((2, HC, 64), jnp.float32),
            pltpu.VMEM((2, HC, 64), jnp.float32),
            pltpu.VMEM((2, HC, 64), jnp.float32),
            pltpu.VMEM((2, HC, 64), jnp.float32),
            pltpu.VMEM((2, HC, 32), jnp.float32),
            pltpu.VMEM((2, HC, 32), jnp.float32),
            pltpu.VMEM((2, CHUNK, 64), jnp.float32),
            pltpu.SemaphoreType.DMA,
            pltpu.SemaphoreType.DMA,
            pltpu.SemaphoreType.DMA,
            pltpu.SemaphoreType.DMA,
        ],
    )
    def k(idx0_h, idx1_h, idx2_h, p_h, b_h, z_h,
          ib0l, ib0h, ib1l, ib1h, ib2l, ib2h,
          bp0l, bp0h, bp2l, bp2h, bbl, bbh, zb,
          ga, gb, wa, wb):
        wid = lax.axis_index("s") * 2 + lax.axis_index("c")
        lo = wid * HPW
        hi = HOFF + wid * HPW
        pltpu.sync_copy(idx0_h.at[pl.ds(lo, HPW)], ib0l)
        pltpu.sync_copy(idx0_h.at[pl.ds(hi, HPW)], ib0h)
        pltpu.sync_copy(idx1_h.at[pl.ds(lo, HPW)], ib1l)
        pltpu.sync_copy(idx1_h.at[pl.ds(hi, HPW)], ib1h)
        pltpu.sync_copy(idx2_h.at[pl.ds(lo, HPW)], ib2l)
        pltpu.sync_copy(idx2_h.at[pl.ds(hi, HPW)], ib2h)

        gsem = (ga, gb)
        wsem = (wa, wb)

        def issue(c, s):
            o = HC * c
            return (
                pltpu.async_copy(p_h.at[ib0l.at[pl.ds(o, HC)]], bp0l.at[s], gsem[s]),
                pltpu.async_copy(p_h.at[ib0h.at[pl.ds(o, HC)]], bp0h.at[s], gsem[s]),
                pltpu.async_copy(b_h.at[ib1l.at[pl.ds(o, HC)]], bbl.at[s], gsem[s]),
                pltpu.async_copy(b_h.at[ib1h.at[pl.ds(o, HC)]], bbh.at[s], gsem[s]),
                pltpu.async_copy(p_h.at[ib2l.at[pl.ds(o, HC)]], bp2l.at[s], gsem[s]),
                pltpu.async_copy(p_h.at[ib2h.at[pl.ds(o, HC)]], bp2h.at[s], gsem[s]),
            )

        def compute(s):
            halves = ((bp0l.at[s], bbl.at[s], bp2l.at[s], 0),
                      (bp0h.at[s], bbh.at[s], bp2h.at[s], 1))
            zr = zb.at[s]

            def row(r, _):
                for p0r, bbr, p2r, kk in halves:
                    for j in range(4):
                        v = (p0r[r, pl.ds(16 * j, 16)]
                             + bbr[r, pl.ds(16 * (j % 2), 16)]
                             + p2r[r, pl.ds(16 * ((j + 2) % 4), 16)])
                        zr[2 * r + kk, pl.ds(16 * j, 16)] = v
                return 0

            lax.fori_loop(0, HC, row, 0)

        gd = [None, None]
        wd = [None, None]
        zbase = wid * RPW
        gd[0] = issue(0, 0)
        for c in range(NCH):
            s = c & 1
            if c + 1 < NCH:
                gd[1 - s] = issue(c + 1, 1 - s)
            for d in gd[s]:
                d.wait()
            if wd[s] is not None:
                wd[s].wait()
            compute(s)
            wd[s] = pltpu.async_copy(
                zb.at[s], z_h.at[pl.ds(zbase + c * CHUNK, CHUNK)], wsem[s]
            )
        for s in (0, 1):
            if wd[s] is not None:
                wd[s].wait()

    return k(idx0, idx1, idx2, p64, b1t)


def _readout_tc(z128, w2t):
    BMZ = 2048
    nrows = z128.shape[0]

    def body(z_ref, w_ref, o_ref):
        z = jnp.maximum(z_ref[...], 0.0)
        o_ref[...] = lax.dot_general(
            w_ref[...], z, (((1,), (1,)), ((), ())),
            preferred_element_type=jnp.float32,
        )

    return pl.pallas_call(
        body,
        grid=(nrows // BMZ,),
        in_specs=[
            pl.BlockSpec((BMZ, 128), lambda i: (i, 0)),
            pl.BlockSpec((6, 128), lambda i: (0, 0)),
        ],
        out_specs=pl.BlockSpec((6, BMZ), lambda i: (0, i)),
        out_shape=jax.ShapeDtypeStruct((6, nrows), jnp.float32),
    )(z128, w2t)


def kernel(x, frag_idx, W1, b1, Wout, bout):
    G = jnp.concatenate([W1[:D_FEAT], W1[2 * D_FEAT:], W1[D_FEAT:2 * D_FEAT]], axis=1)
    bias96 = jnp.concatenate([jnp.zeros((64,), jnp.float32), b1])[None, :]

    idx_pad = jnp.concatenate(
        [frag_idx, jnp.zeros((3, NPAD - N_FRAG), jnp.int32)], axis=1
    )

    p64, b1t = _tables_tc(x, G, bias96)
    z = _gather_sc(idx_pad[0], idx_pad[1], idx_pad[2], p64, b1t)

    m = jnp.concatenate([Wout, Wout], axis=0)
    w2t = jnp.kron(jnp.eye(2, dtype=jnp.float32), m).T
    w2t = w2t[jnp.array([0, 3, 1, 4, 2, 5])]

    z128 = z.reshape(NPAD // 2, 128)
    out_t = _readout_tc(z128, w2t)
    eq_t = out_t.reshape(OUT_DIM, NPAD)
    return eq_t.T[:N_FRAG] + bout[None, :]
